# baseline (speedup 1.0000x reference)
import functools

import jax
import jax.numpy as jnp
from jax import lax
from jax.experimental import pallas as pl
from jax.experimental.pallas import tpu as pltpu
from jax.experimental.pallas import tpu_sc as plsc

N = 10000
E = 320000
D = 128
H = 32

NC = 2
NS = 16
CH = 128
RPT = 624
RPT_LAST = N - 15 * RPT

EPT1 = E // NS
NF1 = EPT1 // CH
TAIL1 = EPT1 - NF1 * CH

EPT2 = E // (NC * NS)
NF2 = EPT2 // CH
TAIL2 = EPT2 - NF2 * CH


def _zero_accum(zeros_hbm, accum, s):
    row0 = pl.multiple_of(s * RPT, 8)

    @pl.when(s < NS - 1)
    def _():
        pltpu.sync_copy(zeros_hbm.at[pl.ds(row0, RPT)],
                        accum.at[pl.ds(row0, RPT)])

    @pl.when(s == NS - 1)
    def _():
        pltpu.sync_copy(zeros_hbm.at[pl.ds(row0, RPT_LAST)],
                        accum.at[pl.ds(row0, RPT_LAST)])


def _readout(accum, out_hbm, s, slab):
    row0 = pl.multiple_of(s * RPT, 8)
    out0 = pl.multiple_of(slab * N + s * RPT, 8)

    @pl.when(s < NS - 1)
    def _():
        pltpu.sync_copy(accum.at[pl.ds(row0, RPT)],
                        out_hbm.at[pl.ds(out0, RPT)])

    @pl.when(s == NS - 1)
    def _():
        pltpu.sync_copy(accum.at[pl.ds(row0, RPT_LAST)],
                        out_hbm.at[pl.ds(out0, RPT_LAST)])


def _edge_loop(y_hbm, src2d, dst2d, accum, rows, semg, sems, nf):
    def issue_gather(j, b):
        pltpu.async_copy(y_hbm.at[src2d.at[j]], rows[b], semg[b])

    def wait_gather(j, b):
        pltpu.make_async_copy(y_hbm.at[src2d.at[j]], rows[b], semg[b]).wait()

    def issue_scatter(j, b):
        pltpu.async_copy(rows[b], accum.at[dst2d.at[j]], sems[b], add=True)

    def wait_scatter(j, b):
        pltpu.make_async_copy(rows[b], accum.at[dst2d.at[j]],
                              sems[b]).wait()

    issue_gather(0, 0)
    issue_gather(1, 1)

    def body(jj, carry):
        for b in (0, 1, 2):
            j = 3 * jj + b
            wait_gather(j, b)

            @pl.when(j + 2 < nf)
            def _():
                @pl.when(j >= 1)
                def _():
                    wait_scatter(j - 1, (b + 2) % 3)
                issue_gather(j + 2, (b + 2) % 3)

            issue_scatter(j, b)
        return carry

    lax.fori_loop(0, nf // 3, body, 0)

    for b in (0, 1, 2):
        wait_scatter(nf - 3 + b, b)


def _tail_chunk(y_hbm, src_hbm, dst_hbm, accum, srcv_t, dstv_t, rows_t, sem,
                src_off, dst_off):
    pltpu.sync_copy(src_hbm.at[pl.ds(src_off, srcv_t.shape[0])], srcv_t)
    pltpu.sync_copy(dst_hbm.at[pl.ds(dst_off, dstv_t.shape[0])], dstv_t)
    pltpu.async_copy(y_hbm.at[srcv_t], rows_t, sem).wait()
    pltpu.sync_copy(rows_t, accum.at[dstv_t], add=True)


def _sc_scratch(W, nf, tail):
    return [
        pltpu.VMEM((nf, CH), jnp.int32),
        pltpu.VMEM((nf, CH), jnp.int32),
        pltpu.VMEM((CH, W), jnp.float32),
        pltpu.VMEM((CH, W), jnp.float32),
        pltpu.VMEM((CH, W), jnp.float32),
        pltpu.VMEM((tail,), jnp.int32),
        pltpu.VMEM((tail,), jnp.int32),
        pltpu.VMEM((tail, W), jnp.float32),
        pltpu.VMEM_SHARED((N, W), jnp.float32),
        pltpu.SemaphoreType.DMA,
        pltpu.SemaphoreType.DMA,
        pltpu.SemaphoreType.DMA,
        pltpu.SemaphoreType.DMA,
        pltpu.SemaphoreType.DMA,
        pltpu.SemaphoreType.DMA,
        pltpu.SemaphoreType.DMA,
    ]


_MESH = plsc.VectorSubcoreMesh(core_axis_name="c", subcore_axis_name="s")
_SC_PARAMS = pltpu.CompilerParams(use_tc_tiling_on_sc=False)


@functools.partial(
    pl.kernel,
    out_type=jax.ShapeDtypeStruct((2 * N, D // 2), jnp.float32),
    mesh=_MESH,
    scratch_types=_sc_scratch(D // 2, NF1, TAIL1),
    compiler_params=_SC_PARAMS,
)
def _sc_agg1(y_hbm, src3d_hbm, dst3d_hbm, src_hbm, dst_hbm, zeros_hbm,
             out_hbm, src2d, dst2d, r0, r1, r2, srcv_t, dstv_t, rows_t,
             accum, semi, sg0, sg1, sg2, ss0, ss1, ss2):
    c = lax.axis_index("c")
    s = lax.axis_index("s")
    pltpu.async_copy(src3d_hbm.at[c * NS + s], src2d, semi)
    pltpu.async_copy(dst3d_hbm.at[s], dst2d, semi)
    _zero_accum(zeros_hbm, accum, s)
    plsc.subcore_barrier()
    pltpu.make_async_copy(src3d_hbm.at[0], src2d, semi).wait()
    pltpu.make_async_copy(dst3d_hbm.at[0], dst2d, semi).wait()
    _edge_loop(y_hbm, src2d, dst2d, accum, (r0, r1, r2),
               (sg0, sg1, sg2), (ss0, ss1, ss2), NF1)
    _tail_chunk(y_hbm, src_hbm, dst_hbm, accum, srcv_t, dstv_t, rows_t, sg0,
                pl.multiple_of(c * E + s * EPT1 + NF1 * CH, 8),
                pl.multiple_of(s * EPT1 + NF1 * CH, 8))
    plsc.subcore_barrier()
    _readout(accum, out_hbm, s, c)


@functools.partial(
    pl.kernel,
    out_type=jax.ShapeDtypeStruct((2 * N, H), jnp.float32),
    mesh=_MESH,
    scratch_types=_sc_scratch(H, NF2, TAIL2),
    compiler_params=_SC_PARAMS,
)
def _sc_agg2(y_hbm, src3d_hbm, dst3d_hbm, src_hbm, dst_hbm, zeros_hbm,
             out_hbm, src2d, dst2d, r0, r1, r2, srcv_t, dstv_t, rows_t,
             accum, semi, sg0, sg1, sg2, ss0, ss1, ss2):
    c = lax.axis_index("c")
    s = lax.axis_index("s")
    w = c * NS + s
    pltpu.async_copy(src3d_hbm.at[w], src2d, semi)
    pltpu.async_copy(dst3d_hbm.at[w], dst2d, semi)
    _zero_accum(zeros_hbm, accum, s)
    plsc.subcore_barrier()
    pltpu.make_async_copy(src3d_hbm.at[0], src2d, semi).wait()
    pltpu.make_async_copy(dst3d_hbm.at[0], dst2d, semi).wait()
    _edge_loop(y_hbm, src2d, dst2d, accum, (r0, r1, r2),
               (sg0, sg1, sg2), (ss0, ss1, ss2), NF2)
    _tail_chunk(y_hbm, src_hbm, dst_hbm, accum, srcv_t, dstv_t, rows_t, sg0,
                pl.multiple_of(w * EPT2 + NF2 * CH, 8),
                pl.multiple_of(w * EPT2 + NF2 * CH, 8))
    plsc.subcore_barrier()
    _readout(accum, out_hbm, s, c)


def _tc_conv1(x, aggL, aggR, W1a, b1a, W1b, b1b):
    def body(x_ref, al_ref, ar_ref, wa_ref, ba_ref, wb_ref, bb_ref, o_ref):
        agg = jnp.concatenate([al_ref[...], ar_ref[...]], axis=1)
        h = x_ref[...] + agg
        u = jnp.maximum(jnp.dot(h, wa_ref[...],
                                preferred_element_type=jnp.float32)
                        + ba_ref[...], 0.0)
        v = jnp.dot(u, wb_ref[...], preferred_element_type=jnp.float32)
        o_ref[...] = jnp.maximum(v + bb_ref[...], 0.0)
    return pl.pallas_call(
        body, out_shape=jax.ShapeDtypeStruct((N, H), jnp.float32))(
            x, aggL, aggR, W1a, b1a.reshape(1, H), W1b, b1b.reshape(1, H))


def _tc_conv2(h1, q0, q1, W2a, b2a, W2b, b2b, Wf, bf):
    def body(h_ref, q0_ref, q1_ref, wa_ref, ba_ref, wb_ref, bb_ref,
             wf_ref, bf_ref, o_ref):
        g = h_ref[...] + (q0_ref[...] + q1_ref[...])
        u = jnp.maximum(jnp.dot(g, wa_ref[...],
                                preferred_element_type=jnp.float32)
                        + ba_ref[...], 0.0)
        v = jnp.dot(u, wb_ref[...], preferred_element_type=jnp.float32)
        h2 = jnp.maximum(v + bb_ref[...], 0.0)
        o_ref[...] = jnp.dot(h2, wf_ref[...],
                             preferred_element_type=jnp.float32) + bf_ref[...]
    return pl.pallas_call(
        body, out_shape=jax.ShapeDtypeStruct((N, 1), jnp.float32))(
            h1, q0, q1, W2a, b2a.reshape(1, H), W2b, b2b.reshape(1, H),
            Wf, bf.reshape(1, 1))


def kernel(x, edge_index, W1a, b1a, W1b, b1b, W2a, b2a, W2b, b2b, Wf, bf):
    src = edge_index[0]
    dst = edge_index[1]
    x2 = x.reshape(2 * N, D // 2)
    src_eo = jnp.concatenate([src * 2, src * 2 + 1])
    zeros_d = jnp.zeros((N, D // 2), jnp.float32)
    zeros_h = jnp.zeros((N, H), jnp.float32)

    src3d_1 = src_eo.reshape(NC * NS, EPT1)[:, :NF1 * CH].reshape(
        NC * NS, NF1, CH)
    dst3d_1 = dst.reshape(NS, EPT1)[:, :NF1 * CH].reshape(NS, NF1, CH)
    src3d_2 = src.reshape(NC * NS, EPT2)[:, :NF2 * CH].reshape(
        NC * NS, NF2, CH)
    dst3d_2 = dst.reshape(NC * NS, EPT2)[:, :NF2 * CH].reshape(
        NC * NS, NF2, CH)

    agg1 = _sc_agg1(x2, src3d_1, dst3d_1, src_eo, dst, zeros_d)
    h1 = _tc_conv1(x, agg1[:N], agg1[N:], W1a, b1a, W1b, b1b)
    parts2 = _sc_agg2(h1, src3d_2, dst3d_2, src, dst, zeros_h)
    return _tc_conv2(h1, parts2[:N], parts2[N:], W2a, b2a, W2b, b2b, Wf, bf)


# speedup vs baseline: 1.1555x; 1.1555x over previous
import functools

import jax
import jax.numpy as jnp
from jax import lax
from jax.experimental import pallas as pl
from jax.experimental.pallas import tpu as pltpu
from jax.experimental.pallas import tpu_sc as plsc

N = 10000
E = 320000
D = 128
H = 32

NC = 2
NS = 16
CH = 128
RPT = 624
RPT_LAST = N - 15 * RPT

EPT1 = E // NS
NF1 = EPT1 // CH
TAIL1 = EPT1 - NF1 * CH

EPT2 = E // (NC * NS)
NF2 = EPT2 // CH
TAIL2 = EPT2 - NF2 * CH


def _zero_accum(zeros_hbm, accum, s):
    row0 = pl.multiple_of(s * RPT, 8)

    @pl.when(s < NS - 1)
    def _():
        pltpu.sync_copy(zeros_hbm.at[pl.ds(row0, RPT)],
                        accum.at[pl.ds(row0, RPT)])

    @pl.when(s == NS - 1)
    def _():
        pltpu.sync_copy(zeros_hbm.at[pl.ds(row0, RPT_LAST)],
                        accum.at[pl.ds(row0, RPT_LAST)])


def _readout(accum, out_hbm, s, col0, W):
    row0 = pl.multiple_of(s * RPT, 8)
    c0 = pl.multiple_of(col0, 8)

    @pl.when(s < NS - 1)
    def _():
        pltpu.sync_copy(accum.at[pl.ds(row0, RPT)],
                        out_hbm.at[pl.ds(row0, RPT), pl.ds(c0, W)])

    @pl.when(s == NS - 1)
    def _():
        pltpu.sync_copy(accum.at[pl.ds(row0, RPT_LAST)],
                        out_hbm.at[pl.ds(row0, RPT_LAST), pl.ds(c0, W)])


def _edge_loop(y_hbm, src2d, dst2d, accum, rows, semg, sems, nf):
    def issue_gather(j, b):
        pltpu.async_copy(y_hbm.at[src2d.at[j]], rows[b], semg[b])

    def wait_gather(j, b):
        pltpu.make_async_copy(y_hbm.at[src2d.at[j]], rows[b], semg[b]).wait()

    def issue_scatter(j, b):
        pltpu.async_copy(rows[b], accum.at[dst2d.at[j]], sems[b], add=True)

    def wait_scatter(j, b):
        pltpu.make_async_copy(rows[b], accum.at[dst2d.at[j]],
                              sems[b]).wait()

    issue_gather(0, 0)
    issue_gather(1, 1)

    def body(jj, carry):
        for b in (0, 1, 2):
            j = 3 * jj + b
            wait_gather(j, b)

            @pl.when(j + 2 < nf)
            def _():
                @pl.when(j >= 1)
                def _():
                    wait_scatter(j - 1, (b + 2) % 3)
                issue_gather(j + 2, (b + 2) % 3)

            issue_scatter(j, b)
        return carry

    lax.fori_loop(0, nf // 3, body, 0)

    for b in (0, 1, 2):
        wait_scatter(nf - 3 + b, b)


def _tail_chunk(y_hbm, src_hbm, dst_hbm, accum, srcv_t, dstv_t, rows_t, sem,
                src_off, dst_off):
    pltpu.sync_copy(src_hbm.at[pl.ds(src_off, srcv_t.shape[0])], srcv_t)
    pltpu.sync_copy(dst_hbm.at[pl.ds(dst_off, dstv_t.shape[0])], dstv_t)
    pltpu.async_copy(y_hbm.at[srcv_t], rows_t, sem).wait()
    pltpu.sync_copy(rows_t, accum.at[dstv_t], add=True)


def _sc_scratch(W, nf, tail):
    return [
        pltpu.VMEM((nf, CH), jnp.int32),
        pltpu.VMEM((nf, CH), jnp.int32),
        pltpu.VMEM((CH, W), jnp.float32),
        pltpu.VMEM((CH, W), jnp.float32),
        pltpu.VMEM((CH, W), jnp.float32),
        pltpu.VMEM((tail,), jnp.int32),
        pltpu.VMEM((tail,), jnp.int32),
        pltpu.VMEM((tail, W), jnp.float32),
        pltpu.VMEM_SHARED((N, W), jnp.float32),
        pltpu.SemaphoreType.DMA,
        pltpu.SemaphoreType.DMA,
        pltpu.SemaphoreType.DMA,
        pltpu.SemaphoreType.DMA,
        pltpu.SemaphoreType.DMA,
        pltpu.SemaphoreType.DMA,
        pltpu.SemaphoreType.DMA,
    ]


_MESH = plsc.VectorSubcoreMesh(core_axis_name="c", subcore_axis_name="s")
_SC_PARAMS = pltpu.CompilerParams(use_tc_tiling_on_sc=False)


@functools.partial(
    pl.kernel,
    out_type=jax.ShapeDtypeStruct((N, D), jnp.float32),
    mesh=_MESH,
    scratch_types=_sc_scratch(D // 2, NF1, TAIL1),
    compiler_params=_SC_PARAMS,
)
def _sc_agg1(y_hbm, src3d_hbm, dst3d_hbm, src_hbm, dst_hbm, zeros_hbm,
             out_hbm, src2d, dst2d, r0, r1, r2, srcv_t, dstv_t, rows_t,
             accum, semi, sg0, sg1, sg2, ss0, ss1, ss2):
    c = lax.axis_index("c")
    s = lax.axis_index("s")
    pltpu.async_copy(src3d_hbm.at[c * NS + s], src2d, semi)
    pltpu.async_copy(dst3d_hbm.at[s], dst2d, semi)
    _zero_accum(zeros_hbm, accum, s)
    plsc.subcore_barrier()
    pltpu.make_async_copy(src3d_hbm.at[0], src2d, semi).wait()
    pltpu.make_async_copy(dst3d_hbm.at[0], dst2d, semi).wait()
    _edge_loop(y_hbm, src2d, dst2d, accum, (r0, r1, r2),
               (sg0, sg1, sg2), (ss0, ss1, ss2), NF1)
    _tail_chunk(y_hbm, src_hbm, dst_hbm, accum, srcv_t, dstv_t, rows_t, sg0,
                pl.multiple_of(c * E + s * EPT1 + NF1 * CH, 8),
                pl.multiple_of(s * EPT1 + NF1 * CH, 8))
    plsc.subcore_barrier()
    _readout(accum, out_hbm, s, c * (D // 2), D // 2)


@functools.partial(
    pl.kernel,
    out_type=jax.ShapeDtypeStruct((N, D), jnp.float32),
    mesh=_MESH,
    scratch_types=_sc_scratch(H, NF2, TAIL2),
    compiler_params=_SC_PARAMS,
)
def _sc_agg2(y_hbm, src3d_hbm, dst3d_hbm, src_hbm, dst_hbm, zeros_hbm,
             out_hbm, src2d, dst2d, r0, r1, r2, srcv_t, dstv_t, rows_t,
             accum, semi, sg0, sg1, sg2, ss0, ss1, ss2):
    c = lax.axis_index("c")
    s = lax.axis_index("s")
    w = c * NS + s
    pltpu.async_copy(src3d_hbm.at[w], src2d, semi)
    pltpu.async_copy(dst3d_hbm.at[w], dst2d, semi)
    _zero_accum(zeros_hbm, accum, s)
    plsc.subcore_barrier()
    pltpu.make_async_copy(src3d_hbm.at[0], src2d, semi).wait()
    pltpu.make_async_copy(dst3d_hbm.at[0], dst2d, semi).wait()
    _edge_loop(y_hbm, src2d, dst2d, accum, (r0, r1, r2),
               (sg0, sg1, sg2), (ss0, ss1, ss2), NF2)
    _tail_chunk(y_hbm, src_hbm, dst_hbm, accum, srcv_t, dstv_t, rows_t, sg0,
                pl.multiple_of(w * EPT2 + NF2 * CH, 8),
                pl.multiple_of(w * EPT2 + NF2 * CH, 8))
    plsc.subcore_barrier()
    _readout(accum, out_hbm, s, c * H, H)


def _tc_conv1(x, agg, W1a, b1a, W1b, b1b):
    def body(x_ref, agg_ref, wa_ref, ba_ref, wb_ref, bb_ref, o_ref):
        h = x_ref[...] + agg_ref[...]
        u = jnp.maximum(jnp.dot(h, wa_ref[...],
                                preferred_element_type=jnp.float32)
                        + ba_ref[...], 0.0)
        v = jnp.dot(u, wb_ref[...], preferred_element_type=jnp.float32)
        o_ref[...] = jnp.maximum(v + bb_ref[...], 0.0)
    return pl.pallas_call(
        body, out_shape=jax.ShapeDtypeStruct((N, H), jnp.float32))(
            x, agg, W1a, b1a.reshape(1, H), W1b, b1b.reshape(1, H))


def _tc_conv2(h1, parts2, W2a, b2a, W2b, b2b, Wf, bf):
    def body(h_ref, p_ref, wa_ref, ba_ref, wb_ref, bb_ref,
             wf_ref, bf_ref, o_ref):
        p = p_ref[...]
        g = h_ref[...] + (p[:, :H] + p[:, H:2 * H])
        u = jnp.maximum(jnp.dot(g, wa_ref[...],
                                preferred_element_type=jnp.float32)
                        + ba_ref[...], 0.0)
        v = jnp.dot(u, wb_ref[...], preferred_element_type=jnp.float32)
        h2 = jnp.maximum(v + bb_ref[...], 0.0)
        o_ref[...] = jnp.dot(h2, wf_ref[...],
                             preferred_element_type=jnp.float32) + bf_ref[...]
    return pl.pallas_call(
        body, out_shape=jax.ShapeDtypeStruct((N, 1), jnp.float32))(
            h1, parts2, W2a, b2a.reshape(1, H), W2b, b2b.reshape(1, H),
            Wf, bf.reshape(1, 1))


def kernel(x, edge_index, W1a, b1a, W1b, b1b, W2a, b2a, W2b, b2b, Wf, bf):
    src = edge_index[0]
    dst = edge_index[1]
    x2 = x.reshape(2 * N, D // 2)
    src_eo = jnp.concatenate([src * 2, src * 2 + 1])
    zeros_d = jnp.zeros((N, D // 2), jnp.float32)
    zeros_h = jnp.zeros((N, H), jnp.float32)

    src3d_1 = src_eo.reshape(NC * NS, EPT1)[:, :NF1 * CH].reshape(
        NC * NS, NF1, CH)
    dst3d_1 = dst.reshape(NS, EPT1)[:, :NF1 * CH].reshape(NS, NF1, CH)
    src3d_2 = src.reshape(NC * NS, EPT2)[:, :NF2 * CH].reshape(
        NC * NS, NF2, CH)
    dst3d_2 = dst.reshape(NC * NS, EPT2)[:, :NF2 * CH].reshape(
        NC * NS, NF2, CH)

    agg1 = _sc_agg1(x2, src3d_1, dst3d_1, src_eo, dst, zeros_d)
    h1 = _tc_conv1(x, agg1, W1a, b1a, W1b, b1b)
    parts2 = _sc_agg2(h1, src3d_2, dst3d_2, src, dst, zeros_h)
    return _tc_conv2(h1, parts2, W2a, b2a, W2b, b2b, Wf, bf)


# speedup vs baseline: 1.2433x; 1.0760x over previous
import functools

import jax
import jax.numpy as jnp
from jax import lax
from jax.experimental import pallas as pl
from jax.experimental.pallas import tpu as pltpu
from jax.experimental.pallas import tpu_sc as plsc

N = 10000
E = 320000
D = 128
H = 32

NC = 2
NS = 16
CH = 128
CH2 = 256
RPT = 624
RPT_LAST = N - 15 * RPT

EPT1 = E // NS
NF1 = EPT1 // CH
TAIL1 = EPT1 - NF1 * CH

EPT2 = E // (NC * NS)
NF2 = EPT2 // CH2
TAIL2 = EPT2 - NF2 * CH2


def _zero_accum(zeros_hbm, accum, s):
    row0 = pl.multiple_of(s * RPT, 8)

    @pl.when(s < NS - 1)
    def _():
        pltpu.sync_copy(zeros_hbm.at[pl.ds(row0, RPT)],
                        accum.at[pl.ds(row0, RPT)])

    @pl.when(s == NS - 1)
    def _():
        pltpu.sync_copy(zeros_hbm.at[pl.ds(row0, RPT_LAST)],
                        accum.at[pl.ds(row0, RPT_LAST)])


def _readout(accum, out_hbm, s, col0, W):
    row0 = pl.multiple_of(s * RPT, 8)
    c0 = pl.multiple_of(col0, 8)

    @pl.when(s < NS - 1)
    def _():
        pltpu.sync_copy(accum.at[pl.ds(row0, RPT)],
                        out_hbm.at[pl.ds(row0, RPT), pl.ds(c0, W)])

    @pl.when(s == NS - 1)
    def _():
        pltpu.sync_copy(accum.at[pl.ds(row0, RPT_LAST)],
                        out_hbm.at[pl.ds(row0, RPT_LAST), pl.ds(c0, W)])


def _edge_loop(y_hbm, src2d, dst2d, accum, rows, semg, sems, nf):
    def issue_gather(j, b):
        pltpu.async_copy(y_hbm.at[src2d.at[j]], rows[b], semg[b])

    def wait_gather(j, b):
        pltpu.make_async_copy(y_hbm.at[src2d.at[j]], rows[b], semg[b]).wait()

    def issue_scatter(j, b):
        pltpu.async_copy(rows[b], accum.at[dst2d.at[j]], sems[b], add=True)

    def wait_scatter(j, b):
        pltpu.make_async_copy(rows[b], accum.at[dst2d.at[j]],
                              sems[b]).wait()

    issue_gather(0, 0)
    issue_gather(1, 1)

    def body(jj, carry):
        for b in (0, 1, 2):
            j = 3 * jj + b
            wait_gather(j, b)

            @pl.when(j + 2 < nf)
            def _():
                @pl.when(j >= 1)
                def _():
                    wait_scatter(j - 1, (b + 2) % 3)
                issue_gather(j + 2, (b + 2) % 3)

            issue_scatter(j, b)
        return carry

    lax.fori_loop(0, nf // 3, body, 0)

    for b in (0, 1, 2):
        wait_scatter(nf - 3 + b, b)


def _tail_chunk(y_hbm, src_hbm, dst_hbm, accum, srcv_t, dstv_t, rows_t, sem,
                src_off, dst_off):
    pltpu.sync_copy(src_hbm.at[pl.ds(src_off, srcv_t.shape[0])], srcv_t)
    pltpu.sync_copy(dst_hbm.at[pl.ds(dst_off, dstv_t.shape[0])], dstv_t)
    pltpu.async_copy(y_hbm.at[srcv_t], rows_t, sem).wait()
    pltpu.sync_copy(rows_t, accum.at[dstv_t], add=True)


def _sc_scratch(W, nf, tail, ch):
    return [
        pltpu.VMEM((nf, ch), jnp.int32),
        pltpu.VMEM((nf, ch), jnp.int32),
        pltpu.VMEM((ch, W), jnp.float32),
        pltpu.VMEM((ch, W), jnp.float32),
        pltpu.VMEM((ch, W), jnp.float32),
        pltpu.VMEM((tail,), jnp.int32),
        pltpu.VMEM((tail,), jnp.int32),
        pltpu.VMEM((tail, W), jnp.float32),
        pltpu.VMEM_SHARED((N, W), jnp.float32),
        pltpu.SemaphoreType.DMA,
        pltpu.SemaphoreType.DMA,
        pltpu.SemaphoreType.DMA,
        pltpu.SemaphoreType.DMA,
        pltpu.SemaphoreType.DMA,
        pltpu.SemaphoreType.DMA,
        pltpu.SemaphoreType.DMA,
    ]


_MESH = plsc.VectorSubcoreMesh(core_axis_name="c", subcore_axis_name="s")
_SC_PARAMS = pltpu.CompilerParams(use_tc_tiling_on_sc=False)


@functools.partial(
    pl.kernel,
    out_type=jax.ShapeDtypeStruct((N, D), jnp.float32),
    mesh=_MESH,
    scratch_types=_sc_scratch(D // 2, NF1, TAIL1, CH),
    compiler_params=_SC_PARAMS,
)
def _sc_agg1(y_hbm, src3d_hbm, dst3d_hbm, src_hbm, dst_hbm, zeros_hbm,
             out_hbm, src2d, dst2d, r0, r1, r2, srcv_t, dstv_t, rows_t,
             accum, semi, sg0, sg1, sg2, ss0, ss1, ss2):
    c = lax.axis_index("c")
    s = lax.axis_index("s")
    pltpu.async_copy(src3d_hbm.at[c * NS + s], src2d, semi)
    pltpu.async_copy(dst3d_hbm.at[s], dst2d, semi)
    _zero_accum(zeros_hbm, accum, s)
    plsc.subcore_barrier()
    pltpu.make_async_copy(src3d_hbm.at[0], src2d, semi).wait()
    pltpu.make_async_copy(dst3d_hbm.at[0], dst2d, semi).wait()
    _edge_loop(y_hbm, src2d, dst2d, accum, (r0, r1, r2),
               (sg0, sg1, sg2), (ss0, ss1, ss2), NF1)
    _tail_chunk(y_hbm, src_hbm, dst_hbm, accum, srcv_t, dstv_t, rows_t, sg0,
                pl.multiple_of(c * E + s * EPT1 + NF1 * CH, 8),
                pl.multiple_of(s * EPT1 + NF1 * CH, 8))
    plsc.subcore_barrier()
    _readout(accum, out_hbm, s, c * (D // 2), D // 2)


@functools.partial(
    pl.kernel,
    out_type=jax.ShapeDtypeStruct((N, D), jnp.float32),
    mesh=_MESH,
    scratch_types=_sc_scratch(H, NF2, TAIL2, CH2),
    compiler_params=_SC_PARAMS,
)
def _sc_agg2(y_hbm, src3d_hbm, dst3d_hbm, src_hbm, dst_hbm, zeros_hbm,
             out_hbm, src2d, dst2d, r0, r1, r2, srcv_t, dstv_t, rows_t,
             accum, semi, sg0, sg1, sg2, ss0, ss1, ss2):
    c = lax.axis_index("c")
    s = lax.axis_index("s")
    w = c * NS + s
    pltpu.async_copy(src3d_hbm.at[w], src2d, semi)
    pltpu.async_copy(dst3d_hbm.at[w], dst2d, semi)
    _zero_accum(zeros_hbm, accum, s)
    plsc.subcore_barrier()
    pltpu.make_async_copy(src3d_hbm.at[0], src2d, semi).wait()
    pltpu.make_async_copy(dst3d_hbm.at[0], dst2d, semi).wait()
    _edge_loop(y_hbm, src2d, dst2d, accum, (r0, r1, r2),
               (sg0, sg1, sg2), (ss0, ss1, ss2), NF2)
    _tail_chunk(y_hbm, src_hbm, dst_hbm, accum, srcv_t, dstv_t, rows_t, sg0,
                pl.multiple_of(w * EPT2 + NF2 * CH2, 8),
                pl.multiple_of(w * EPT2 + NF2 * CH2, 8))
    plsc.subcore_barrier()
    _readout(accum, out_hbm, s, c * H, H)


def _tc_conv1(x, agg, W1a, b1a, W1b, b1b):
    def body(x_ref, agg_ref, wa_ref, ba_ref, wb_ref, bb_ref, o_ref):
        h = x_ref[...] + agg_ref[...]
        u = jnp.maximum(jnp.dot(h, wa_ref[...],
                                preferred_element_type=jnp.float32)
                        + ba_ref[...], 0.0)
        v = jnp.dot(u, wb_ref[...], preferred_element_type=jnp.float32)
        o_ref[...] = jnp.maximum(v + bb_ref[...], 0.0)
    return pl.pallas_call(
        body, out_shape=jax.ShapeDtypeStruct((N, H), jnp.float32))(
            x, agg, W1a, b1a.reshape(1, H), W1b, b1b.reshape(1, H))


def _tc_conv2(h1, parts2, W2a, b2a, W2b, b2b, Wf, bf):
    def body(h_ref, p_ref, wa_ref, ba_ref, wb_ref, bb_ref,
             wf_ref, bf_ref, o_ref):
        p = p_ref[...]
        g = h_ref[...] + (p[:, :H] + p[:, H:2 * H])
        u = jnp.maximum(jnp.dot(g, wa_ref[...],
                                preferred_element_type=jnp.float32)
                        + ba_ref[...], 0.0)
        v = jnp.dot(u, wb_ref[...], preferred_element_type=jnp.float32)
        h2 = jnp.maximum(v + bb_ref[...], 0.0)
        o_ref[...] = jnp.dot(h2, wf_ref[...],
                             preferred_element_type=jnp.float32) + bf_ref[...]
    return pl.pallas_call(
        body, out_shape=jax.ShapeDtypeStruct((N, 1), jnp.float32))(
            h1, parts2, W2a, b2a.reshape(1, H), W2b, b2b.reshape(1, H),
            Wf, bf.reshape(1, 1))


def kernel(x, edge_index, W1a, b1a, W1b, b1b, W2a, b2a, W2b, b2b, Wf, bf):
    src = edge_index[0]
    dst = edge_index[1]
    x2 = x.reshape(2 * N, D // 2)
    src_eo = jnp.concatenate([src * 2, src * 2 + 1])
    zeros_d = jnp.zeros((N, D // 2), jnp.float32)
    zeros_h = jnp.zeros((N, H), jnp.float32)

    src3d_1 = src_eo.reshape(NC * NS, EPT1)[:, :NF1 * CH].reshape(
        NC * NS, NF1, CH)
    dst3d_1 = dst.reshape(NS, EPT1)[:, :NF1 * CH].reshape(NS, NF1, CH)
    src3d_2 = src.reshape(NC * NS, EPT2)[:, :NF2 * CH2].reshape(
        NC * NS, NF2, CH2)
    dst3d_2 = dst.reshape(NC * NS, EPT2)[:, :NF2 * CH2].reshape(
        NC * NS, NF2, CH2)

    agg1 = _sc_agg1(x2, src3d_1, dst3d_1, src_eo, dst, zeros_d)
    h1 = _tc_conv1(x, agg1, W1a, b1a, W1b, b1b)
    parts2 = _sc_agg2(h1, src3d_2, dst3d_2, src, dst, zeros_h)
    return _tc_conv2(h1, parts2, W2a, b2a, W2b, b2b, Wf, bf)
